# Initial kernel scaffold; baseline (speedup 1.0000x reference)
import jax
import jax.numpy as jnp
from jax.experimental import pallas as pl


def kernel(src_points, ref_points, log_n_affinity):
    raise NotImplementedError("write your pallas kernel here")


# speedup vs baseline: 9.9557x; 9.9557x over previous
import jax
import jax.numpy as jnp
from jax.experimental import pallas as pl
from jax.experimental.pallas import tpu as pltpu

N_SS = 1024
N_RR = 1024
K_OUT = 512
BS = 8
INT_MAX = 2147483647


def _knn_body(pts_ref, ptsT_ref, idx_out, dist_out):
    d2 = None
    for c in range(3):
        a = pts_ref[:, c:c + 1]
        b = ptsT_ref[c:c + 1, :]
        e = (a - b) * (a - b)
        d2 = e if d2 is None else d2 + e
    dist = jnp.sqrt(jnp.maximum(d2, 0.0))
    ciota = jax.lax.broadcasted_iota(jnp.int32, dist.shape, 1)
    liota = jax.lax.broadcasted_iota(jnp.int32, (dist.shape[0], 8), 1)
    iacc = jnp.zeros((dist.shape[0], 8), jnp.int32)
    dacc = jnp.zeros((dist.shape[0], 8), jnp.float32)
    for k in range(6):
        m = jnp.min(dist, axis=1, keepdims=True)
        cand = jnp.where(dist == m, ciota, INT_MAX)
        sel = jnp.min(cand, axis=1, keepdims=True)
        iacc = jnp.where(liota == k, sel, iacc)
        dacc = jnp.where(liota == k, m, dacc)
        dist = jnp.where(ciota == sel, jnp.inf, dist)
    idx_out[...] = iacc
    dist_out[...] = dacc


def _knn(points):
    ptsT = jnp.zeros((8, N_SS), jnp.float32).at[:3, :].set(points.T)
    return pl.pallas_call(
        _knn_body,
        grid=(8,),
        in_specs=[
            pl.BlockSpec((128, 3), lambda i: (i, 0)),
            pl.BlockSpec((8, N_SS), lambda i: (0, 0)),
        ],
        out_specs=[
            pl.BlockSpec((128, 8), lambda i: (i, 0)),
            pl.BlockSpec((128, 8), lambda i: (i, 0)),
        ],
        out_shape=[
            jax.ShapeDtypeStruct((N_SS, 8), jnp.int32),
            jax.ShapeDtypeStruct((N_SS, 8), jnp.float32),
        ],
    )(points, ptsT)


def _gather_body(rT_ref, AT_ref, out_ref):
    pid = pl.program_id(0)

    def body(k, carry):
        idx = rT_ref[pid * 64 + k]
        out_ref[pl.ds(k, 1), :] = AT_ref[pl.ds(idx, 1), :]
        return carry

    jax.lax.fori_loop(0, 64, body, 0)


def _gather_cols(AT, rT_flat):
    grid_spec = pltpu.PrefetchScalarGridSpec(
        num_scalar_prefetch=1,
        grid=(5 * N_RR // 64,),
        in_specs=[pl.BlockSpec((N_SS, N_SS), lambda i, *_: (0, 0))],
        out_specs=pl.BlockSpec((64, N_SS), lambda i, *_: (i, 0)),
    )
    return pl.pallas_call(
        _gather_body,
        grid_spec=grid_spec,
        out_shape=jax.ShapeDtypeStruct((5 * N_RR, N_SS), jnp.float32),
    )(rT_flat, AT)


def _main_body(sidx_ref, A_ref, AGs_ref, ds_ref, drT_ref,
               vals_out, idxs_out, G_scr, vals_scr, idxs_scr):
    pid = pl.program_id(0)

    @pl.when(pid == 0)
    def _init():
        vals_scr[...] = jnp.full((4, 128), -1.0, jnp.float32)
        idxs_scr[...] = jnp.zeros((4, 128), jnp.int32)

    riota = jax.lax.broadcasted_iota(jnp.int32, (BS, N_RR), 0)
    ciota = jax.lax.broadcasted_iota(jnp.int32, (BS, N_RR), 1)
    fbase = (pid * BS + riota) * (N_RR * 25) + ciota * 25
    b0 = jax.lax.broadcasted_iota(jnp.int32, (4, 128), 0)
    b1 = jax.lax.broadcasted_iota(jnp.int32, (4, 128), 1)
    biota = b0 * 128 + b1

    A_blk = A_ref[...]

    for i in range(5):
        for s in range(BS):
            G_scr[s] = AGs_ref[sidx_ref[pid * BS + s, i + 1]]
        dscol = ds_ref[:, i + 1:i + 2]
        for j in range(5):
            C = G_scr[:, j, :]
            drrow = drT_ref[j + 1:j + 2, :]
            diff = dscol - drrow
            t = jnp.maximum(1.0 - (diff * diff) / (0.1 ** 2), 0.0)
            P = (A_blk * t) * C
            fmat = fbase + (i * 5 + j)

            def _minkey():
                bv = vals_scr[...]
                bi = idxs_scr[...]
                bmin = jnp.min(bv)
                bidx = jnp.max(jnp.where(bv == bmin, bi, -1))
                return bmin, bidx

            def _tilemax(P_):
                m_ = jnp.max(P_)
                fsel_ = jnp.min(jnp.where(P_ == m_, fmat, INT_MAX))
                return m_, fsel_

            m, fsel = _tilemax(P)
            bmin0, bidx0 = _minkey()

            def cond(carry):
                _, m_, fsel_, bmin_, bidx_ = carry
                return (m_ > bmin_) | ((m_ == bmin_) & (fsel_ < bidx_))

            def body(carry):
                P_, m_, fsel_, bmin_, bidx_ = carry
                bv = vals_scr[...]
                bi = idxs_scr[...]
                bpos = jnp.min(jnp.where((bv == bmin_) & (bi == bidx_),
                                         biota, INT_MAX))
                vals_scr[...] = jnp.where(biota == bpos, m_, bv)
                idxs_scr[...] = jnp.where(biota == bpos, fsel_, bi)
                P_ = jnp.where(fmat == fsel_, -1.0, P_)
                m2, f2 = _tilemax(P_)
                bmin2, bidx2 = _minkey()
                return (P_, m2, f2, bmin2, bidx2)

            jax.lax.while_loop(cond, body, (P, m, fsel, bmin0, bidx0))

    vals_out[...] = vals_scr[...]
    idxs_out[...] = idxs_scr[...]


def _main(sidx, A, AGs, ds, drT):
    grid_spec = pltpu.PrefetchScalarGridSpec(
        num_scalar_prefetch=1,
        grid=(N_SS // BS,),
        in_specs=[
            pl.BlockSpec((BS, N_RR), lambda i, *_: (i, 0)),
            pl.BlockSpec((N_SS, 5, N_RR), lambda i, *_: (0, 0, 0)),
            pl.BlockSpec((BS, 8), lambda i, *_: (i, 0)),
            pl.BlockSpec((8, N_RR), lambda i, *_: (0, 0)),
        ],
        out_specs=[
            pl.BlockSpec((4, 128), lambda i, *_: (0, 0)),
            pl.BlockSpec((4, 128), lambda i, *_: (0, 0)),
        ],
        scratch_shapes=[
            pltpu.VMEM((BS, 5, N_RR), jnp.float32),
            pltpu.VMEM((4, 128), jnp.float32),
            pltpu.VMEM((4, 128), jnp.int32),
        ],
    )
    return pl.pallas_call(
        _main_body,
        grid_spec=grid_spec,
        out_shape=[
            jax.ShapeDtypeStruct((4, 128), jnp.float32),
            jax.ShapeDtypeStruct((4, 128), jnp.int32),
        ],
    )(sidx, A, AGs, ds, drT)


def kernel(src_points, ref_points, log_n_affinity):
    n_r = N_RR
    sidx8, sdist8 = _knn(src_points)
    ridx8, rdist8 = _knn(ref_points)

    AT = log_n_affinity.T
    rT_flat = ridx8.T[1:6].reshape(-1)
    AG = _gather_cols(AT, rT_flat)
    AGs = AG.reshape(5, n_r, N_SS).transpose(2, 0, 1)

    drT = rdist8.T
    vals, fidx = _main(sidx8, log_n_affinity, AGs, sdist8, drT)

    vals = vals.reshape(-1)
    flat = fidx.reshape(-1)
    order = jnp.lexsort((flat, -vals))
    flat = flat[order]

    first_node_src = flat // (n_r * 25)
    rem = flat % (n_r * 25)
    first_node_ref = rem // 25
    second_idx = rem % 25
    sls = second_idx // 5
    slr = second_idx % 5
    src_ne_idx = sidx8[:, 1:6]
    ref_ne_idx = ridx8[:, 1:6]
    second_node_src = src_ne_idx[first_node_src, sls]
    second_node_ref = ref_ne_idx[first_node_ref, slr]
    return (first_node_src, first_node_ref, second_node_src, second_node_ref)


# speedup vs baseline: 9.9747x; 1.0019x over previous
import jax
import jax.numpy as jnp
from jax.experimental import pallas as pl
from jax.experimental.pallas import tpu as pltpu

N_SS = 1024
N_RR = 1024
K_OUT = 512
BS = 8
INT_MAX = 2147483647


def _knn_body(pts_ref, ptsT_ref, idx_out, dist_out):
    d2 = None
    for c in range(3):
        a = pts_ref[:, c:c + 1]
        b = ptsT_ref[c:c + 1, :]
        e = (a - b) * (a - b)
        d2 = e if d2 is None else d2 + e
    dist = jnp.sqrt(jnp.maximum(d2, 0.0))
    ciota = jax.lax.broadcasted_iota(jnp.int32, dist.shape, 1)
    liota = jax.lax.broadcasted_iota(jnp.int32, (dist.shape[0], 8), 1)
    iacc = jnp.zeros((dist.shape[0], 8), jnp.int32)
    dacc = jnp.zeros((dist.shape[0], 8), jnp.float32)
    for k in range(6):
        m = jnp.min(dist, axis=1, keepdims=True)
        cand = jnp.where(dist == m, ciota, INT_MAX)
        sel = jnp.min(cand, axis=1, keepdims=True)
        iacc = jnp.where(liota == k, sel, iacc)
        dacc = jnp.where(liota == k, m, dacc)
        dist = jnp.where(ciota == sel, jnp.inf, dist)
    idx_out[...] = iacc
    dist_out[...] = dacc


def _knn(points):
    ptsT = jnp.zeros((8, N_SS), jnp.float32).at[:3, :].set(points.T)
    return pl.pallas_call(
        _knn_body,
        grid=(8,),
        in_specs=[
            pl.BlockSpec((128, 3), lambda i: (i, 0)),
            pl.BlockSpec((8, N_SS), lambda i: (0, 0)),
        ],
        out_specs=[
            pl.BlockSpec((128, 8), lambda i: (i, 0)),
            pl.BlockSpec((128, 8), lambda i: (i, 0)),
        ],
        out_shape=[
            jax.ShapeDtypeStruct((N_SS, 8), jnp.int32),
            jax.ShapeDtypeStruct((N_SS, 8), jnp.float32),
        ],
    )(points, ptsT)


def _gather_body(rT_ref, AT_ref, out_ref):
    pid = pl.program_id(0)

    def body(k, carry):
        idx = rT_ref[pid * 64 + k]
        out_ref[pl.ds(k, 1), :] = AT_ref[pl.ds(idx, 1), :]
        return carry

    jax.lax.fori_loop(0, 64, body, 0)


def _gather_cols(AT, rT_flat):
    grid_spec = pltpu.PrefetchScalarGridSpec(
        num_scalar_prefetch=1,
        grid=(5 * N_RR // 64,),
        in_specs=[pl.BlockSpec((N_SS, N_SS), lambda i, *_: (0, 0))],
        out_specs=pl.BlockSpec((64, N_SS), lambda i, *_: (i, 0)),
    )
    return pl.pallas_call(
        _gather_body,
        grid_spec=grid_spec,
        out_shape=jax.ShapeDtypeStruct((5 * N_RR, N_SS), jnp.float32),
    )(rT_flat, AT)


def _main_body(sidx_ref, A_ref, AGs_ref, ds_ref, drT_ref,
               vals_out, idxs_out, G_scr, vals_scr, idxs_scr):
    pid = pl.program_id(0)

    @pl.when(pid == 0)
    def _init():
        vals_scr[...] = jnp.full((4, 128), -1.0, jnp.float32)
        idxs_scr[...] = jnp.zeros((4, 128), jnp.int32)

    riota = jax.lax.broadcasted_iota(jnp.int32, (BS, N_RR), 0)
    ciota = jax.lax.broadcasted_iota(jnp.int32, (BS, N_RR), 1)
    fbase = (pid * BS + riota) * (N_RR * 25) + ciota * 25
    b0 = jax.lax.broadcasted_iota(jnp.int32, (4, 128), 0)
    b1 = jax.lax.broadcasted_iota(jnp.int32, (4, 128), 1)
    biota = b0 * 128 + b1

    A_blk = A_ref[...]

    for i in range(5):
        for s in range(BS):
            idx = sidx_ref[pid * BS + s, i + 1]
            G_scr[pl.ds(s, 1), :] = AGs_ref[pl.ds(idx, 1), :]
        dscol = ds_ref[:, i + 1:i + 2]
        for j in range(5):
            C = G_scr[:, j * N_RR:(j + 1) * N_RR]
            drrow = drT_ref[j + 1:j + 2, :]
            diff = dscol - drrow
            t = jnp.maximum(1.0 - (diff * diff) / (0.1 ** 2), 0.0)
            P = (A_blk * t) * C
            fmat = fbase + (i * 5 + j)

            def _minkey():
                bv = vals_scr[...]
                bi = idxs_scr[...]
                bmin = jnp.min(bv)
                bidx = jnp.max(jnp.where(bv == bmin, bi, -1))
                return bmin, bidx

            def _tilemax(P_):
                m_ = jnp.max(P_)
                fsel_ = jnp.min(jnp.where(P_ == m_, fmat, INT_MAX))
                return m_, fsel_

            m, fsel = _tilemax(P)
            bmin0, bidx0 = _minkey()

            def cond(carry):
                m_, fsel_, bmin_, bidx_ = carry
                return (m_ > bmin_) | ((m_ == bmin_) & (fsel_ < bidx_))

            def body(carry, P=P, fmat=fmat):
                m_, fsel_, bmin_, bidx_ = carry
                bv = vals_scr[...]
                bi = idxs_scr[...]
                bpos = jnp.min(jnp.where((bv == bmin_) & (bi == bidx_),
                                         biota, INT_MAX))
                vals_scr[...] = jnp.where(biota == bpos, m_, bv)
                idxs_scr[...] = jnp.where(biota == bpos, fsel_, bi)
                pred = (P < m_) | ((P == m_) & (fmat > fsel_))
                m2 = jnp.max(jnp.where(pred, P, -1.0))
                f2 = jnp.min(jnp.where(pred & (P == m2), fmat, INT_MAX))
                bmin2, bidx2 = _minkey()
                return (m2, f2, bmin2, bidx2)

            jax.lax.while_loop(cond, body, (m, fsel, bmin0, bidx0))

    vals_out[...] = vals_scr[...]
    idxs_out[...] = idxs_scr[...]


def _main(sidx, A, AGs, ds, drT):
    grid_spec = pltpu.PrefetchScalarGridSpec(
        num_scalar_prefetch=1,
        grid=(N_SS // BS,),
        in_specs=[
            pl.BlockSpec((BS, N_RR), lambda i, *_: (i, 0)),
            pl.BlockSpec((N_SS, 5 * N_RR), lambda i, *_: (0, 0)),
            pl.BlockSpec((BS, 8), lambda i, *_: (i, 0)),
            pl.BlockSpec((8, N_RR), lambda i, *_: (0, 0)),
        ],
        out_specs=[
            pl.BlockSpec((4, 128), lambda i, *_: (0, 0)),
            pl.BlockSpec((4, 128), lambda i, *_: (0, 0)),
        ],
        scratch_shapes=[
            pltpu.VMEM((BS, 5 * N_RR), jnp.float32),
            pltpu.VMEM((4, 128), jnp.float32),
            pltpu.VMEM((4, 128), jnp.int32),
        ],
    )
    return pl.pallas_call(
        _main_body,
        grid_spec=grid_spec,
        out_shape=[
            jax.ShapeDtypeStruct((4, 128), jnp.float32),
            jax.ShapeDtypeStruct((4, 128), jnp.int32),
        ],
    )(sidx, A, AGs, ds, drT)


def kernel(src_points, ref_points, log_n_affinity):
    n_r = N_RR
    sidx8, sdist8 = _knn(src_points)
    ridx8, rdist8 = _knn(ref_points)

    AT = log_n_affinity.T
    rT_flat = ridx8.T[1:6].reshape(-1)
    AG = _gather_cols(AT, rT_flat)
    AGs = AG.T

    drT = rdist8.T
    vals, fidx = _main(sidx8, log_n_affinity, AGs, sdist8, drT)

    vals = vals.reshape(-1)
    flat = fidx.reshape(-1)
    order = jnp.lexsort((flat, -vals))
    flat = flat[order]

    first_node_src = flat // (n_r * 25)
    rem = flat % (n_r * 25)
    first_node_ref = rem // 25
    second_idx = rem % 25
    sls = second_idx // 5
    slr = second_idx % 5
    src_ne_idx = sidx8[:, 1:6]
    ref_ne_idx = ridx8[:, 1:6]
    second_node_src = src_ne_idx[first_node_src, sls]
    second_node_ref = ref_ne_idx[first_node_ref, slr]
    return (first_node_src, first_node_ref, second_node_src, second_node_ref)


# speedup vs baseline: 9.9948x; 1.0020x over previous
import jax
import jax.numpy as jnp
from jax.experimental import pallas as pl
from jax.experimental.pallas import tpu as pltpu

N_SS = 1024
N_RR = 1024
K_OUT = 512
BS = 8
INT_MAX = 2147483647


def _knn_body(pts_ref, ptsT_ref, idx_out, dist_out):
    d2 = None
    for c in range(3):
        a = pts_ref[:, c:c + 1]
        b = ptsT_ref[c:c + 1, :]
        e = (a - b) * (a - b)
        d2 = e if d2 is None else d2 + e
    dist = jnp.sqrt(jnp.maximum(d2, 0.0))
    ciota = jax.lax.broadcasted_iota(jnp.int32, dist.shape, 1)
    liota = jax.lax.broadcasted_iota(jnp.int32, (dist.shape[0], 8), 1)
    iacc = jnp.zeros((dist.shape[0], 8), jnp.int32)
    dacc = jnp.zeros((dist.shape[0], 8), jnp.float32)
    for k in range(6):
        m = jnp.min(dist, axis=1, keepdims=True)
        cand = jnp.where(dist == m, ciota, INT_MAX)
        sel = jnp.min(cand, axis=1, keepdims=True)
        iacc = jnp.where(liota == k, sel, iacc)
        dacc = jnp.where(liota == k, m, dacc)
        dist = jnp.where(ciota == sel, jnp.inf, dist)
    idx_out[...] = iacc
    dist_out[...] = dacc


def _knn(points):
    ptsT = jnp.zeros((8, N_SS), jnp.float32).at[:3, :].set(points.T)
    return pl.pallas_call(
        _knn_body,
        grid=(8,),
        in_specs=[
            pl.BlockSpec((128, 3), lambda i: (i, 0)),
            pl.BlockSpec((8, N_SS), lambda i: (0, 0)),
        ],
        out_specs=[
            pl.BlockSpec((128, 8), lambda i: (i, 0)),
            pl.BlockSpec((128, 8), lambda i: (i, 0)),
        ],
        out_shape=[
            jax.ShapeDtypeStruct((N_SS, 8), jnp.int32),
            jax.ShapeDtypeStruct((N_SS, 8), jnp.float32),
        ],
    )(points, ptsT)


def _gather_body(rT_ref, AT_ref, out_ref):
    pid = pl.program_id(0)

    def body(k, carry):
        idx = rT_ref[pid * 64 + k]
        out_ref[pl.ds(k, 1), :] = AT_ref[pl.ds(idx, 1), :]
        return carry

    jax.lax.fori_loop(0, 64, body, 0)


def _gather_cols(AT, rT_flat):
    grid_spec = pltpu.PrefetchScalarGridSpec(
        num_scalar_prefetch=1,
        grid=(5 * N_RR // 64,),
        in_specs=[pl.BlockSpec((N_SS, N_SS), lambda i, *_: (0, 0))],
        out_specs=pl.BlockSpec((64, N_SS), lambda i, *_: (i, 0)),
    )
    return pl.pallas_call(
        _gather_body,
        grid_spec=grid_spec,
        out_shape=jax.ShapeDtypeStruct((5 * N_RR, N_SS), jnp.float32),
    )(rT_flat, AT)


def _main_body(sidx_ref, A_ref, AGs_ref, ds_ref, drT_ref,
               vals_out, idxs_out, G_scr, P_scr, vals_scr, idxs_scr):
    pid = pl.program_id(0)

    @pl.when(pid == 0)
    def _init():
        vals_scr[...] = jnp.full((4, 128), -1.0, jnp.float32)
        idxs_scr[...] = jnp.zeros((4, 128), jnp.int32)

    riota = jax.lax.broadcasted_iota(jnp.int32, (BS, N_RR), 0)
    ciota = jax.lax.broadcasted_iota(jnp.int32, (BS, N_RR), 1)
    fbase = (pid * BS + riota) * (N_RR * 25) + ciota * 25
    b0 = jax.lax.broadcasted_iota(jnp.int32, (4, 128), 0)
    b1 = jax.lax.broadcasted_iota(jnp.int32, (4, 128), 1)
    biota = b0 * 128 + b1

    A_blk = A_ref[...]

    mstep = None
    for i in range(5):
        for s in range(BS):
            idx = sidx_ref[pid * BS + s, i + 1]
            G_scr[pl.ds(s, 1), :] = AGs_ref[pl.ds(idx, 1), :]
        dscol = ds_ref[:, i + 1:i + 2]
        for j in range(5):
            C = G_scr[:, j * N_RR:(j + 1) * N_RR]
            drrow = drT_ref[j + 1:j + 2, :]
            diff = dscol - drrow
            t = jnp.maximum(1.0 - (diff * diff) / (0.1 ** 2), 0.0)
            P = (A_blk * t) * C
            P_scr[:, (i * 5 + j) * N_RR:(i * 5 + j + 1) * N_RR] = P
            mstep = P if mstep is None else jnp.maximum(mstep, P)

    def _minkey():
        bv = vals_scr[...]
        bi = idxs_scr[...]
        bmin = jnp.min(bv)
        bidx = jnp.max(jnp.where(bv == bmin, bi, -1))
        return bmin, bidx

    def _process_tile(i, j):
        P = P_scr[:, (i * 5 + j) * N_RR:(i * 5 + j + 1) * N_RR]
        fmat = fbase + (i * 5 + j)
        m = jnp.max(P)
        fsel = jnp.min(jnp.where(P == m, fmat, INT_MAX))
        bmin0, bidx0 = _minkey()

        def cond(carry):
            m_, fsel_, bmin_, bidx_ = carry
            return (m_ > bmin_) | ((m_ == bmin_) & (fsel_ < bidx_))

        def body(carry):
            m_, fsel_, bmin_, bidx_ = carry
            bv = vals_scr[...]
            bi = idxs_scr[...]
            bpos = jnp.min(jnp.where((bv == bmin_) & (bi == bidx_),
                                     biota, INT_MAX))
            vals_scr[...] = jnp.where(biota == bpos, m_, bv)
            idxs_scr[...] = jnp.where(biota == bpos, fsel_, bi)
            pred = (P < m_) | ((P == m_) & (fmat > fsel_))
            m2 = jnp.max(jnp.where(pred, P, -1.0))
            f2 = jnp.min(jnp.where(pred & (P == m2), fmat, INT_MAX))
            bmin2, bidx2 = _minkey()
            return (m2, f2, bmin2, bidx2)

        jax.lax.while_loop(cond, body, (m, fsel, bmin0, bidx0))

    gate = jnp.max(mstep) >= jnp.min(vals_scr[...])

    @pl.when(gate)
    def _hot():
        for i in range(5):
            for j in range(5):
                _process_tile(i, j)

    vals_out[...] = vals_scr[...]
    idxs_out[...] = idxs_scr[...]


def _main(sidx, A, AGs, ds, drT):
    grid_spec = pltpu.PrefetchScalarGridSpec(
        num_scalar_prefetch=1,
        grid=(N_SS // BS,),
        in_specs=[
            pl.BlockSpec((BS, N_RR), lambda i, *_: (i, 0)),
            pl.BlockSpec((N_SS, 5 * N_RR), lambda i, *_: (0, 0)),
            pl.BlockSpec((BS, 8), lambda i, *_: (i, 0)),
            pl.BlockSpec((8, N_RR), lambda i, *_: (0, 0)),
        ],
        out_specs=[
            pl.BlockSpec((4, 128), lambda i, *_: (0, 0)),
            pl.BlockSpec((4, 128), lambda i, *_: (0, 0)),
        ],
        scratch_shapes=[
            pltpu.VMEM((BS, 5 * N_RR), jnp.float32),
            pltpu.VMEM((BS, 25 * N_RR), jnp.float32),
            pltpu.VMEM((4, 128), jnp.float32),
            pltpu.VMEM((4, 128), jnp.int32),
        ],
    )
    return pl.pallas_call(
        _main_body,
        grid_spec=grid_spec,
        out_shape=[
            jax.ShapeDtypeStruct((4, 128), jnp.float32),
            jax.ShapeDtypeStruct((4, 128), jnp.int32),
        ],
    )(sidx, A, AGs, ds, drT)


def kernel(src_points, ref_points, log_n_affinity):
    n_r = N_RR
    sidx8, sdist8 = _knn(src_points)
    ridx8, rdist8 = _knn(ref_points)

    AT = log_n_affinity.T
    rT_flat = ridx8.T[1:6].reshape(-1)
    AG = _gather_cols(AT, rT_flat)
    AGs = AG.T

    drT = rdist8.T
    vals, fidx = _main(sidx8, log_n_affinity, AGs, sdist8, drT)

    vals = vals.reshape(-1)
    flat = fidx.reshape(-1)
    order = jnp.lexsort((flat, -vals))
    flat = flat[order]

    first_node_src = flat // (n_r * 25)
    rem = flat % (n_r * 25)
    first_node_ref = rem // 25
    second_idx = rem % 25
    sls = second_idx // 5
    slr = second_idx % 5
    src_ne_idx = sidx8[:, 1:6]
    ref_ne_idx = ridx8[:, 1:6]
    second_node_src = src_ne_idx[first_node_src, sls]
    second_node_ref = ref_ne_idx[first_node_ref, slr]
    return (first_node_src, first_node_ref, second_node_src, second_node_ref)
